# baseline (speedup 1.0000x reference)
import functools

import jax
import jax.numpy as jnp
from jax import lax
from jax.experimental import pallas as pl
from jax.experimental.pallas import tpu as pltpu
from jax.experimental.pallas import tpu_sc as plsc

N = 10000
E = 320000
D = 128

NC = 2
NS = 16
NW = NC * NS

EDGES_PER_W = E // NW
CHUNK = 40
NCHUNK = EDGES_PER_W // CHUNK
NBUF = 5
NSUPER = NCHUNK // NBUF
STRIPE = 624
LAST_STRIPE = N - STRIPE * (NS - 1)

DEG_W = 16


def _make_sc_agg(with_deg: bool):
    acc_t = jax.ShapeDtypeStruct((NC * N, D), jnp.float32)
    dacc_t = jax.ShapeDtypeStruct((NC * N, DEG_W), jnp.float32)
    out_type = (acc_t, dacc_t) if with_deg else acc_t

    scratch = (
        [pltpu.VMEM((CHUNK,), jnp.int32) for _ in range(4 * NBUF)]
        + [pltpu.VMEM((CHUNK, D), jnp.float32) for _ in range(NBUF)]
        + [pltpu.VMEM_SHARED((N, D), jnp.float32)]
        + ([pltpu.VMEM((CHUNK, DEG_W), jnp.float32),
            pltpu.VMEM_SHARED((N, DEG_W), jnp.float32)]
           if with_deg else [])
        + [pltpu.SemaphoreType.DMA for _ in range(4 * NBUF)]
    )

    mesh = plsc.VectorSubcoreMesh(core_axis_name="c", subcore_axis_name="s")

    @functools.partial(
        pl.kernel,
        out_type=out_type,
        mesh=mesh,
        scratch_types=scratch,
        compiler_params=pltpu.CompilerParams(use_tc_tiling_on_sc=False),
    )
    def sc_agg(feat, src, dst, zrows, zdeg, ones_h, *out_and_scratch):
        if with_deg:
            acc_out, dacc_out = out_and_scratch[0], out_and_scratch[1]
            rest = out_and_scratch[2:]
        else:
            acc_out = out_and_scratch[0]
            dacc_out = None
            rest = out_and_scratch[1:]
        idx_s = (rest[0:NBUF], rest[NBUF:2 * NBUF])
        idx_d = (rest[2 * NBUF:3 * NBUF], rest[3 * NBUF:4 * NBUF])
        rows = rest[4 * NBUF:5 * NBUF]
        k = 5 * NBUF
        acc_sh = rest[k]
        k += 1
        if with_deg:
            ones_v, dacc_sh = rest[k:k + 2]
            k += 2
        else:
            ones_v = dacc_sh = None
        semI = (rest[k:k + NBUF], rest[k + NBUF:k + 2 * NBUF])
        semG = rest[k + 2 * NBUF:k + 3 * NBUF]
        semS = rest[k + 3 * NBUF:k + 4 * NBUF]

        c = lax.axis_index("c")
        s = lax.axis_index("s")
        w = c * NS + s

        r0 = pl.multiple_of(s * STRIPE, 8)
        ebase = w * EDGES_PER_W

        def issue_idx(g, p, b):
            base = pl.multiple_of(ebase + (g * NBUF + b) * CHUNK, 8)
            pltpu.async_copy(src.at[pl.ds(base, CHUNK)], idx_s[p][b], semI[p][b])
            pltpu.async_copy(dst.at[pl.ds(base, CHUNK)], idx_d[p][b], semI[p][b])

        for b in range(NBUF):
            issue_idx(0, 0, b)
            issue_idx(1, 1, b)

        @pl.when(s < NS - 1)
        def _():
            pltpu.sync_copy(zrows.at[pl.ds(0, STRIPE)], acc_sh.at[pl.ds(r0, STRIPE)])
            if with_deg:
                pltpu.sync_copy(zdeg.at[pl.ds(0, STRIPE)],
                                dacc_sh.at[pl.ds(r0, STRIPE)])

        @pl.when(s == NS - 1)
        def _():
            pltpu.sync_copy(zrows, acc_sh.at[pl.ds(N - LAST_STRIPE, LAST_STRIPE)])
            if with_deg:
                pltpu.sync_copy(zdeg, dacc_sh.at[pl.ds(N - LAST_STRIPE, LAST_STRIPE)])

        if with_deg:
            pltpu.sync_copy(ones_h, ones_v)

        def group(g, p, first):
            q = 1 - p
            gathers = []
            for b in range(NBUF):
                if not first:
                    pltpu.make_async_copy(rows[b], acc_sh.at[idx_d[q][b]],
                                          semS[b]).wait()
                    if with_deg:
                        pltpu.make_async_copy(ones_v, dacc_sh.at[idx_d[q][b]],
                                              semS[b]).wait()

                    @pl.when(g < NSUPER - 1)
                    def _(b=b):
                        issue_idx(g + 1, q, b)

                pltpu.make_async_copy(src.at[pl.ds(0, CHUNK)], idx_s[p][b],
                                      semI[p][b]).wait()
                pltpu.make_async_copy(dst.at[pl.ds(0, CHUNK)], idx_d[p][b],
                                      semI[p][b]).wait()
                if with_deg:
                    pltpu.async_copy(ones_v, dacc_sh.at[idx_d[p][b]],
                                     semS[b], add=True)
                gathers.append(pltpu.async_copy(feat.at[idx_s[p][b]], rows[b],
                                                semG[b]))
            for b in range(NBUF):
                gathers[b].wait()
                pltpu.async_copy(rows[b], acc_sh.at[idx_d[p][b]],
                                 semS[b], add=True)

        gathers0 = []
        for b in range(NBUF):
            pltpu.make_async_copy(src.at[pl.ds(0, CHUNK)], idx_s[0][b],
                                  semI[0][b]).wait()
            pltpu.make_async_copy(dst.at[pl.ds(0, CHUNK)], idx_d[0][b],
                                  semI[0][b]).wait()
            gathers0.append(pltpu.async_copy(feat.at[idx_s[0][b]], rows[b],
                                             semG[b]))
        plsc.subcore_barrier()
        for b in range(NBUF):
            if with_deg:
                pltpu.async_copy(ones_v, dacc_sh.at[idx_d[0][b]],
                                 semS[b], add=True)
            gathers0[b].wait()
            pltpu.async_copy(rows[b], acc_sh.at[idx_d[0][b]],
                             semS[b], add=True)

        def pair_step(i, carry):
            group(2 * i + 1, 1, first=False)
            group(2 * i + 2, 0, first=False)
            return carry

        lax.fori_loop(0, (NSUPER - 2) // 2, pair_step, 0)
        group(NSUPER - 1, 1, first=False)

        for b in range(NBUF):
            pltpu.make_async_copy(rows[b], acc_sh.at[idx_d[1][b]],
                                  semS[b]).wait()
            if with_deg:
                pltpu.make_async_copy(ones_v, dacc_sh.at[idx_d[1][b]],
                                      semS[b]).wait()
        plsc.subcore_barrier()

        @pl.when(s < NS - 1)
        def _():
            obase = pl.multiple_of(c * N + r0, 8)
            pltpu.sync_copy(acc_sh.at[pl.ds(r0, STRIPE)],
                            acc_out.at[pl.ds(obase, STRIPE)])
            if with_deg:
                pltpu.sync_copy(dacc_sh.at[pl.ds(r0, STRIPE)],
                                dacc_out.at[pl.ds(obase, STRIPE)])

        @pl.when(s == NS - 1)
        def _():
            lb = N - LAST_STRIPE
            obase = pl.multiple_of(c * N + lb, 8)
            pltpu.sync_copy(acc_sh.at[pl.ds(lb, LAST_STRIPE)],
                            acc_out.at[pl.ds(obase, LAST_STRIPE)])
            if with_deg:
                pltpu.sync_copy(dacc_sh.at[pl.ds(lb, LAST_STRIPE)],
                                dacc_out.at[pl.ds(obase, LAST_STRIPE)])

    return sc_agg


_sc_agg_deg = _make_sc_agg(True)
_sc_agg = _make_sc_agg(False)

BN = 1000


def _tc_body(relu, x_ref, acc_ref, dacc_ref, w_ref, b_ref, o_ref):
    deg = dacc_ref[0, :, 0:1] + dacc_ref[1, :, 0:1]
    invd = 1.0 / jnp.maximum(deg, 1.0)
    agg = (acc_ref[0] + acc_ref[1]) * invd
    h = (jnp.dot(x_ref[...], w_ref[0:D], preferred_element_type=jnp.float32)
         + jnp.dot(agg, w_ref[D:2 * D], preferred_element_type=jnp.float32)
         + b_ref[...])
    o_ref[...] = jnp.maximum(h, 0.0) if relu else h


def _tc_layer(x, acc, dacc, w, b, relu):
    return pl.pallas_call(
        functools.partial(_tc_body, relu),
        grid=(N // BN,),
        in_specs=[
            pl.BlockSpec((BN, D), lambda i: (i, 0)),
            pl.BlockSpec((NC, BN, D), lambda i: (0, i, 0)),
            pl.BlockSpec((NC, BN, DEG_W), lambda i: (0, i, 0)),
            pl.BlockSpec((2 * D, D), lambda i: (0, 0)),
            pl.BlockSpec((1, D), lambda i: (0, 0)),
        ],
        out_specs=pl.BlockSpec((BN, D), lambda i: (i, 0)),
        out_shape=jax.ShapeDtypeStruct((N, D), jnp.float32),
    )(x, acc, dacc, w, b)


def kernel(x, edge_index, W1, b1, W2, b2):
    src = edge_index[0]
    dst = edge_index[1]

    zrows = jnp.zeros((LAST_STRIPE, D), jnp.float32)
    zdeg = jnp.zeros((LAST_STRIPE, DEG_W), jnp.float32)
    ones_h = jnp.ones((CHUNK, DEG_W), jnp.float32)

    acc1, dacc1 = _sc_agg_deg(x, src, dst, zrows, zdeg, ones_h)
    acc1 = acc1.reshape(NC, N, D)
    dacc = dacc1.reshape(NC, N, DEG_W)

    h = _tc_layer(x, acc1, dacc, W1, b1.reshape(1, D), relu=True)

    acc2, _ = _sc_agg_deg(h, src, dst, zrows, zdeg, ones_h)
    acc2 = acc2.reshape(NC, N, D)

    out = _tc_layer(h, acc2, dacc, W2, b2.reshape(1, D), relu=False)
    return out


# speedup vs baseline: 1.0242x; 1.0242x over previous
import functools

import jax
import jax.numpy as jnp
from jax import lax
from jax.experimental import pallas as pl
from jax.experimental.pallas import tpu as pltpu
from jax.experimental.pallas import tpu_sc as plsc

N = 10000
E = 320000
D = 128

NC = 2
NS = 16
NW = NC * NS

EDGES_PER_W = E // NW
CHUNK = 40
NCHUNK = EDGES_PER_W // CHUNK
NBUF = 5
NSUPER = NCHUNK // NBUF
STRIPE = 624
LAST_STRIPE = N - STRIPE * (NS - 1)

DEG_W = 16


def _make_sc_agg(with_deg: bool):
    acc_t = jax.ShapeDtypeStruct((NC * N, D), jnp.float32)
    dacc_t = jax.ShapeDtypeStruct((NC * N, DEG_W), jnp.float32)
    out_type = (acc_t, dacc_t) if with_deg else acc_t

    scratch = (
        [pltpu.VMEM((CHUNK,), jnp.int32) for _ in range(4 * NBUF)]
        + [pltpu.VMEM((CHUNK, D), jnp.float32) for _ in range(NBUF)]
        + [pltpu.VMEM_SHARED((N, D), jnp.float32)]
        + ([pltpu.VMEM((CHUNK, DEG_W), jnp.float32),
            pltpu.VMEM_SHARED((N, DEG_W), jnp.float32)]
           if with_deg else [])
        + [pltpu.SemaphoreType.DMA for _ in range(4 * NBUF)]
    )

    mesh = plsc.VectorSubcoreMesh(core_axis_name="c", subcore_axis_name="s")

    @functools.partial(
        pl.kernel,
        out_type=out_type,
        mesh=mesh,
        scratch_types=scratch,
        compiler_params=pltpu.CompilerParams(use_tc_tiling_on_sc=False),
    )
    def sc_agg(feat, src, dst, zrows, zdeg, ones_h, *out_and_scratch):
        if with_deg:
            acc_out, dacc_out = out_and_scratch[0], out_and_scratch[1]
            rest = out_and_scratch[2:]
        else:
            acc_out = out_and_scratch[0]
            dacc_out = None
            rest = out_and_scratch[1:]
        idx_s = (rest[0:NBUF], rest[NBUF:2 * NBUF])
        idx_d = (rest[2 * NBUF:3 * NBUF], rest[3 * NBUF:4 * NBUF])
        rows = rest[4 * NBUF:5 * NBUF]
        k = 5 * NBUF
        acc_sh = rest[k]
        k += 1
        if with_deg:
            ones_v, dacc_sh = rest[k:k + 2]
            k += 2
        else:
            ones_v = dacc_sh = None
        semI = (rest[k:k + NBUF], rest[k + NBUF:k + 2 * NBUF])
        semG = rest[k + 2 * NBUF:k + 3 * NBUF]
        semS = rest[k + 3 * NBUF:k + 4 * NBUF]

        c = lax.axis_index("c")
        s = lax.axis_index("s")
        w = c * NS + s

        r0 = pl.multiple_of(s * STRIPE, 8)
        ebase = w * EDGES_PER_W

        def issue_idx(g, p, b):
            base = pl.multiple_of(ebase + (g * NBUF + b) * CHUNK, 8)
            pltpu.async_copy(src.at[pl.ds(base, CHUNK)], idx_s[p][b], semI[p][b])
            pltpu.async_copy(dst.at[pl.ds(base, CHUNK)], idx_d[p][b], semI[p][b])

        for b in range(NBUF):
            issue_idx(0, 0, b)
            issue_idx(1, 1, b)

        @pl.when(s < NS - 1)
        def _():
            pltpu.sync_copy(zrows.at[pl.ds(0, STRIPE)], acc_sh.at[pl.ds(r0, STRIPE)])
            if with_deg:
                pltpu.sync_copy(zdeg.at[pl.ds(0, STRIPE)],
                                dacc_sh.at[pl.ds(r0, STRIPE)])

        @pl.when(s == NS - 1)
        def _():
            pltpu.sync_copy(zrows, acc_sh.at[pl.ds(N - LAST_STRIPE, LAST_STRIPE)])
            if with_deg:
                pltpu.sync_copy(zdeg, dacc_sh.at[pl.ds(N - LAST_STRIPE, LAST_STRIPE)])

        if with_deg:
            pltpu.sync_copy(ones_h, ones_v)

        def group(g, p, first):
            q = 1 - p
            gathers = []
            for b in range(NBUF):
                if not first:
                    pltpu.make_async_copy(rows[b], acc_sh.at[idx_d[q][b]],
                                          semS[b]).wait()
                    if with_deg:
                        pltpu.make_async_copy(ones_v, dacc_sh.at[idx_d[q][b]],
                                              semS[b]).wait()

                    @pl.when(g < NSUPER - 1)
                    def _(b=b):
                        issue_idx(g + 1, q, b)

                pltpu.make_async_copy(src.at[pl.ds(0, CHUNK)], idx_s[p][b],
                                      semI[p][b]).wait()
                pltpu.make_async_copy(dst.at[pl.ds(0, CHUNK)], idx_d[p][b],
                                      semI[p][b]).wait()
                if with_deg:
                    pltpu.async_copy(ones_v, dacc_sh.at[idx_d[p][b]],
                                     semS[b], add=True)
                gathers.append(pltpu.async_copy(feat.at[idx_s[p][b]], rows[b],
                                                semG[b]))
            for b in range(NBUF):
                gathers[b].wait()
                pltpu.async_copy(rows[b], acc_sh.at[idx_d[p][b]],
                                 semS[b], add=True)

        gathers0 = []
        for b in range(NBUF):
            pltpu.make_async_copy(src.at[pl.ds(0, CHUNK)], idx_s[0][b],
                                  semI[0][b]).wait()
            pltpu.make_async_copy(dst.at[pl.ds(0, CHUNK)], idx_d[0][b],
                                  semI[0][b]).wait()
            gathers0.append(pltpu.async_copy(feat.at[idx_s[0][b]], rows[b],
                                             semG[b]))
        plsc.subcore_barrier()
        for b in range(NBUF):
            if with_deg:
                pltpu.async_copy(ones_v, dacc_sh.at[idx_d[0][b]],
                                 semS[b], add=True)
            gathers0[b].wait()
            pltpu.async_copy(rows[b], acc_sh.at[idx_d[0][b]],
                             semS[b], add=True)

        def pair_step(i, carry):
            group(2 * i + 1, 1, first=False)
            group(2 * i + 2, 0, first=False)
            return carry

        lax.fori_loop(0, (NSUPER - 2) // 2, pair_step, 0)
        group(NSUPER - 1, 1, first=False)

        for b in range(NBUF):
            pltpu.make_async_copy(rows[b], acc_sh.at[idx_d[1][b]],
                                  semS[b]).wait()
            if with_deg:
                pltpu.make_async_copy(ones_v, dacc_sh.at[idx_d[1][b]],
                                      semS[b]).wait()
        plsc.subcore_barrier()

        @pl.when(s < NS - 1)
        def _():
            obase = pl.multiple_of(c * N + r0, 8)
            pltpu.sync_copy(acc_sh.at[pl.ds(r0, STRIPE)],
                            acc_out.at[pl.ds(obase, STRIPE)])
            if with_deg:
                pltpu.sync_copy(dacc_sh.at[pl.ds(r0, STRIPE)],
                                dacc_out.at[pl.ds(obase, STRIPE)])

        @pl.when(s == NS - 1)
        def _():
            lb = N - LAST_STRIPE
            obase = pl.multiple_of(c * N + lb, 8)
            pltpu.sync_copy(acc_sh.at[pl.ds(lb, LAST_STRIPE)],
                            acc_out.at[pl.ds(obase, LAST_STRIPE)])
            if with_deg:
                pltpu.sync_copy(dacc_sh.at[pl.ds(lb, LAST_STRIPE)],
                                dacc_out.at[pl.ds(obase, LAST_STRIPE)])

    return sc_agg


_sc_agg_deg = _make_sc_agg(True)
_sc_agg = _make_sc_agg(False)

BN = 1000


def _tc_body(relu, x_ref, acc_ref, dacc_ref, w_ref, b_ref, o_ref):
    deg = dacc_ref[0, :, 0:1] + dacc_ref[1, :, 0:1]
    invd = 1.0 / jnp.maximum(deg, 1.0)
    agg = (acc_ref[0] + acc_ref[1]) * invd
    h = (jnp.dot(x_ref[...], w_ref[0:D], preferred_element_type=jnp.float32)
         + jnp.dot(agg, w_ref[D:2 * D], preferred_element_type=jnp.float32)
         + b_ref[...])
    o_ref[...] = jnp.maximum(h, 0.0) if relu else h


def _tc_layer(x, acc, dacc, w, b, relu):
    return pl.pallas_call(
        functools.partial(_tc_body, relu),
        grid=(N // BN,),
        in_specs=[
            pl.BlockSpec((BN, D), lambda i: (i, 0)),
            pl.BlockSpec((NC, BN, D), lambda i: (0, i, 0)),
            pl.BlockSpec((NC, BN, DEG_W), lambda i: (0, i, 0)),
            pl.BlockSpec((2 * D, D), lambda i: (0, 0)),
            pl.BlockSpec((1, D), lambda i: (0, 0)),
        ],
        out_specs=pl.BlockSpec((BN, D), lambda i: (i, 0)),
        out_shape=jax.ShapeDtypeStruct((N, D), jnp.float32),
    )(x, acc, dacc, w, b)


def kernel(x, edge_index, W1, b1, W2, b2):
    src = edge_index[0]
    dst = edge_index[1]

    zrows = jnp.zeros((LAST_STRIPE, D), jnp.float32)
    zdeg = jnp.zeros((LAST_STRIPE, DEG_W), jnp.float32)
    ones_h = jnp.ones((CHUNK, DEG_W), jnp.float32)

    acc1, dacc1 = _sc_agg_deg(x, src, dst, zrows, zdeg, ones_h)
    acc1 = acc1.reshape(NC, N, D)
    dacc = dacc1.reshape(NC, N, DEG_W)

    h = _tc_layer(x, acc1, dacc, W1, b1.reshape(1, D), relu=True)

    acc2 = _sc_agg(h, src, dst, zrows, zdeg, ones_h)
    acc2 = acc2.reshape(NC, N, D)

    out = _tc_layer(h, acc2, dacc, W2, b2.reshape(1, D), relu=False)
    return out
